# baseline (speedup 1.0000x reference)
import jax
import jax.numpy as jnp
from jax.experimental import pallas as pl
from jax.experimental.pallas import tpu as pltpu

D = 1024
H = 16
HD = 64
T = 256
S = 64
G = 2730
VS = 16384
DEPTH = 6
EPS = 1e-5
BG = 512
NG = 6
BV = 2048
NV = 8
f32 = jnp.float32


def _ln(x, w, b):
    m = jnp.mean(x, axis=-1, keepdims=True)
    v = jnp.mean((x - m) ** 2, axis=-1, keepdims=True)
    return (x - m) / jnp.sqrt(v + EPS) * w + b


def _mmT(a, w):
    return jax.lax.dot_general(
        a, w, (((1,), (1,)), ((), ())),
        preferred_element_type=f32,
        precision=jax.lax.Precision.HIGHEST)


def _embed_body(tok_ref, ti_ref, emb_ref, pos_ref, w_ref, b_ref, o_ref):
    x = emb_ref[...] + pos_ref[...]
    o_ref[...] = _ln(x, w_ref[...], b_ref[...])


def _embed(tok, ti, emb, pos, w, b):
    B = tok.shape[0]
    emb3 = emb.reshape(emb.shape[0], 1, D)
    pos3 = pos.reshape(pos.shape[0], 1, D)
    out = pl.pallas_call(
        _embed_body,
        grid_spec=pltpu.PrefetchScalarGridSpec(
            num_scalar_prefetch=2,
            grid=(B,),
            in_specs=[
                pl.BlockSpec((1, 1, D), lambda i, tok, ti: (tok[i], 0, 0)),
                pl.BlockSpec((1, 1, D), lambda i, tok, ti: (ti[0], 0, 0)),
                pl.BlockSpec((1, D), lambda i, tok, ti: (0, 0)),
                pl.BlockSpec((1, D), lambda i, tok, ti: (0, 0)),
            ],
            out_specs=pl.BlockSpec((1, 1, D), lambda i, tok, ti: (i, 0, 0)),
        ),
        out_shape=jax.ShapeDtypeStruct((B, 1, D), f32),
    )(tok, ti, emb3, pos3, w, b)
    return out.reshape(B, D)


def _sa_body(ti_ref, x_ref, st_ref, wq_ref, wk_ref, wv_ref, wo_ref,
             pw_ref, pb_ref, ow_ref, ob_ref, xo_ref, kv_ref):
    ti = ti_ref[0]
    x = x_ref[...]
    B = x.shape[0]
    h = _ln(x, pw_ref[...], pb_ref[...])
    q = _mmT(h, wq_ref[...])
    k = _mmT(h, wk_ref[...])
    v = _mmT(h, wv_ref[...])
    kv_ref[0:B, :] = k
    kv_ref[B:2 * B, :] = v
    qs = q * (1.0 / (HD ** 0.5))
    K = st_ref[0, 0:B]
    V = st_ref[0, B:2 * B]
    sc_old = (K * qs[:, None, :]).reshape(B, T, H, HD).sum(axis=-1)
    s_new = (k * qs).reshape(B, H, HD).sum(axis=-1)
    tpos = jax.lax.broadcasted_iota(jnp.int32, (B, T, H), 1)
    sc_old = jnp.where(tpos < ti, sc_old, jnp.float32(-1e30))
    m = jnp.maximum(sc_old.max(axis=1), s_new)
    p_old = jnp.exp(sc_old - m[:, None, :])
    p_new = jnp.exp(s_new - m)
    denom = p_old.sum(axis=1) + p_new
    pfull = jnp.broadcast_to(p_old[:, :, :, None], (B, T, H, HD)).reshape(B, T, D)
    ctx = (pfull * V).sum(axis=1)
    ctx = ctx + jnp.broadcast_to(p_new[:, :, None], (B, H, HD)).reshape(B, D) * v
    ctx = ctx / jnp.broadcast_to(denom[:, :, None], (B, H, HD)).reshape(B, D)
    o = _mmT(ctx, wo_ref[...])
    xo_ref[...] = x + _ln(o, ow_ref[...], ob_ref[...])


def _sa(lidx, ti, x, state, wq, wk, wv, wo, pw, pb, ow, ob):
    B = x.shape[0]
    full = lambda _: (0, 0)
    return pl.pallas_call(
        _sa_body,
        grid=(1,),
        in_specs=[
            pl.BlockSpec(memory_space=pltpu.SMEM),
            pl.BlockSpec((B, D), full),
            pl.BlockSpec((1, 2 * B, T, D), lambda _: (lidx, 0, 0, 0)),
            pl.BlockSpec((D, D), full),
            pl.BlockSpec((D, D), full),
            pl.BlockSpec((D, D), full),
            pl.BlockSpec((D, D), full),
            pl.BlockSpec((1, D), full),
            pl.BlockSpec((1, D), full),
            pl.BlockSpec((1, D), full),
            pl.BlockSpec((1, D), full),
        ],
        out_specs=[
            pl.BlockSpec((B, D), full),
            pl.BlockSpec((2 * B, D), full),
        ],
        out_shape=[
            jax.ShapeDtypeStruct((B, D), f32),
            jax.ShapeDtypeStruct((2 * B, D), f32),
        ],
    )(ti, x, state, wq, wk, wv, wo, pw, pb, ow, ob)


def _ca_body(x_ref, enc_ref, mk_ref, wq_ref, wk_ref, wv_ref, wo_ref,
             pw_ref, pb_ref, ow_ref, ob_ref, xo_ref):
    x = x_ref[...]
    B = x.shape[0]
    h = _ln(x, pw_ref[...], pb_ref[...])
    q = _mmT(h, wq_ref[...]) * (1.0 / (HD ** 0.5))
    enc2 = enc_ref[...].reshape(B * S, D)
    K = _mmT(enc2, wk_ref[...]).reshape(B, S, D)
    V = _mmT(enc2, wv_ref[...]).reshape(B, S, D)
    sc = (K * q[:, None, :]).reshape(B, S, H, HD).sum(axis=-1)
    bias = (1.0 - mk_ref[...]) * -1e12
    sc = sc + bias[:, :, None]
    m = sc.max(axis=1)
    p = jnp.exp(sc - m[:, None, :])
    denom = p.sum(axis=1)
    pfull = jnp.broadcast_to(p[:, :, :, None], (B, S, H, HD)).reshape(B, S, D)
    ctx = (pfull * V).sum(axis=1)
    ctx = ctx / jnp.broadcast_to(denom[:, :, None], (B, H, HD)).reshape(B, D)
    o = _mmT(ctx, wo_ref[...])
    xo_ref[...] = x + _ln(o, ow_ref[...], ob_ref[...])


def _ca(x, enc, maskf, wq, wk, wv, wo, pw, pb, ow, ob):
    B = x.shape[0]
    full = lambda _: (0, 0)
    return pl.pallas_call(
        _ca_body,
        grid=(1,),
        in_specs=[
            pl.BlockSpec((B, D), full),
            pl.BlockSpec((B, S, D), lambda _: (0, 0, 0)),
            pl.BlockSpec((B, S), full),
            pl.BlockSpec((D, D), full),
            pl.BlockSpec((D, D), full),
            pl.BlockSpec((D, D), full),
            pl.BlockSpec((D, D), full),
            pl.BlockSpec((1, D), full),
            pl.BlockSpec((1, D), full),
            pl.BlockSpec((1, D), full),
            pl.BlockSpec((1, D), full),
        ],
        out_specs=pl.BlockSpec((B, D), full),
        out_shape=jax.ShapeDtypeStruct((B, D), f32),
    )(x, enc, maskf, wq, wk, wv, wo, pw, pb, ow, ob)


def _glu_a_body(x_ref, w0_ref, b0_ref, fc0_ref, fc1_ref, u_ref):
    z = _ln(x_ref[...], w0_ref[...], b0_ref[...])
    w = _mmT(z, fc0_ref[...])
    wv = _mmT(z, fc1_ref[...])
    gelu = w * 0.5 * (1.0 + jax.lax.erf(w * (2.0 ** -0.5)))
    u_ref[...] = gelu * wv


def _glu_a(x, w0, b0, fc0, fc1):
    B = x.shape[0]
    return pl.pallas_call(
        _glu_a_body,
        grid=(NG,),
        in_specs=[
            pl.BlockSpec((B, D), lambda g: (0, 0)),
            pl.BlockSpec((1, D), lambda g: (0, 0)),
            pl.BlockSpec((1, D), lambda g: (0, 0)),
            pl.BlockSpec((BG, D), lambda g: (g, 0)),
            pl.BlockSpec((BG, D), lambda g: (g, 0)),
        ],
        out_specs=pl.BlockSpec((B, BG), lambda g: (0, g)),
        out_shape=jax.ShapeDtypeStruct((B, G), f32),
    )(x, w0, b0, fc0, fc1)


def _glu_b_body(x_ref, u_ref, w1_ref, b1_ref, fc2_ref, xo_ref):
    z = _ln(u_ref[...], w1_ref[...], b1_ref[...])
    xo_ref[...] = x_ref[...] + _mmT(z, fc2_ref[...])


def _glu_b(x, u, w1, b1, fc2):
    B = x.shape[0]
    full = lambda _: (0, 0)
    return pl.pallas_call(
        _glu_b_body,
        grid=(1,),
        in_specs=[
            pl.BlockSpec((B, D), full),
            pl.BlockSpec((B, G), full),
            pl.BlockSpec((1, G), full),
            pl.BlockSpec((1, G), full),
            pl.BlockSpec((D, G), full),
        ],
        out_specs=pl.BlockSpec((B, D), full),
        out_shape=jax.ShapeDtypeStruct((B, D), f32),
    )(x, u, w1, b1, fc2)


def _lm_body(x_ref, set_ref, fw_ref, fb_ref, w_ref, g_ref, o_ref, acc):
    i = pl.program_id(0)
    zf = _ln(x_ref[...], fw_ref[...], fb_ref[...])
    lb = _mmT(zf, w_ref[...])
    scond = set_ref[2]
    IC = lb.shape[0] // 2
    mixed = lb[0:IC] * (1.0 - scond) + lb[IC:2 * IC] * scond
    acc[:, pl.ds(i * BV, BV)] = mixed

    @pl.when(i == NV - 1)
    def _():
        L = acc[...]
        temp = set_ref[0]
        k = jnp.clip(set_ref[1].astype(jnp.int32), 1, VS)
        top1 = L.max(axis=1, keepdims=True)
        lo0 = L.min(axis=1, keepdims=True)
        hi0 = top1 + jnp.abs(top1) + 1.0

        def body(_, lohi):
            lo, hi = lohi
            mid = lo + (hi - lo) * 0.5
            cnt = (L >= mid).astype(jnp.int32).sum(axis=1, keepdims=True)
            ge = cnt >= k
            return (jnp.where(ge, mid, lo), jnp.where(ge, hi, mid))

        kth, _hi = jax.lax.fori_loop(0, 64, body, (lo0, hi0))
        shifted = (L - top1) / temp
        kept = L >= kth
        probs = jnp.exp(shifted) * kept.astype(f32)
        logp = jnp.where(probs > 0, jnp.log(jnp.maximum(probs, 1e-30)), -1e12)
        score = logp + g_ref[...]
        smax = score.max(axis=1, keepdims=True)
        idx = jax.lax.broadcasted_iota(jnp.int32, score.shape, 1)
        cand = jnp.where(score == smax, idx, VS)
        o_ref[...] = cand.min(axis=1, keepdims=True).reshape(1, -1)


def _lm(x, settings, fw, fb, lm_head, gum):
    B = x.shape[0]
    IC = B // 2
    return pl.pallas_call(
        _lm_body,
        grid=(NV,),
        in_specs=[
            pl.BlockSpec((B, D), lambda i: (0, 0)),
            pl.BlockSpec(memory_space=pltpu.SMEM),
            pl.BlockSpec((1, D), lambda i: (0, 0)),
            pl.BlockSpec((1, D), lambda i: (0, 0)),
            pl.BlockSpec((BV, D), lambda i: (i, 0)),
            pl.BlockSpec((IC, VS), lambda i: (0, 0)),
        ],
        out_specs=pl.BlockSpec((1, IC), lambda i: (0, 0)),
        out_shape=jax.ShapeDtypeStruct((1, IC), jnp.int32),
        scratch_shapes=[pltpu.VMEM((IC, VS), f32)],
    )(x, settings, fw, fb, lm_head, gum)


def _stupd_body(ti_ref, st_ref, kv_ref, o_ref):
    r = ti_ref[0] % 8
    o_ref[...] = st_ref[...]
    o_ref[0, :, pl.ds(r, 1), :] = kv_ref[...][0, :, None, :]


def _state_update(ti, state, kv):
    B2 = kv.shape[1]
    return pl.pallas_call(
        _stupd_body,
        grid_spec=pltpu.PrefetchScalarGridSpec(
            num_scalar_prefetch=1,
            grid=(DEPTH,),
            in_specs=[
                pl.BlockSpec((1, B2, 8, D), lambda l, ti: (l, 0, ti[0] // 8, 0)),
                pl.BlockSpec((1, B2, D), lambda l, ti: (l, 0, 0)),
            ],
            out_specs=pl.BlockSpec((1, B2, 8, D), lambda l, ti: (l, 0, ti[0] // 8, 0)),
        ),
        out_shape=jax.ShapeDtypeStruct(state.shape, state.dtype),
        input_output_aliases={1: 0},
    )(ti, state, kv)


def kernel(settings, attention_mask, encoder_state, attention_state,
           prev_tokens, token_index, params):
    IC = encoder_state.shape[0] // 2
    ti = token_index.astype(jnp.int32)
    tok = jnp.clip(jnp.concatenate([prev_tokens, prev_tokens]), 0,
                   params['embed_tokens'].shape[0] - 1).astype(jnp.int32)
    r2 = lambda a: a.reshape(1, -1)
    x = _embed(tok, ti, params['embed_tokens'], params['embed_positions'],
               r2(params['ln_emb_w']), r2(params['ln_emb_b']))
    maskf = attention_mask.astype(f32)
    kvs = []
    for l in range(DEPTH):
        lp = params['layers'][l]
        x, kv = _sa(l, ti, x, attention_state,
                    lp['sa_q'], lp['sa_k'], lp['sa_v'], lp['sa_o'],
                    r2(lp['pre_sa_ln_w']), r2(lp['pre_sa_ln_b']),
                    r2(lp['sa_ln_w']), r2(lp['sa_ln_b']))
        kvs.append(kv)
        x = _ca(x, encoder_state, maskf,
                lp['ca_q'], lp['ca_k'], lp['ca_v'], lp['ca_o'],
                r2(lp['pre_ca_ln_w']), r2(lp['pre_ca_ln_b']),
                r2(lp['ca_ln_w']), r2(lp['ca_ln_b']))
        u = _glu_a(x, r2(lp['glu_ln0_w']), r2(lp['glu_ln0_b']),
                   lp['glu_fc0'], lp['glu_fc1'])
        x = _glu_b(x, u, r2(lp['glu_ln1_w']), r2(lp['glu_ln1_b']),
                   lp['glu_fc2'])
    kvst = jnp.stack(kvs, axis=0)
    new_state = _state_update(ti, attention_state, kvst)
    gum = jax.random.gumbel(jax.random.key(42), (IC, VS), f32)
    tokens = _lm(x, settings, r2(params['final_ln_w']),
                 r2(params['final_ln_b']), params['lm_head'], gum)
    return tokens.reshape(IC), new_state


# speedup vs baseline: 1.3739x; 1.3739x over previous
import jax
import jax.numpy as jnp
from jax.experimental import pallas as pl
from jax.experimental.pallas import tpu as pltpu

D = 1024
H = 16
HD = 64
T = 256
S = 64
G = 2730
VS = 16384
DEPTH = 6
EPS = 1e-5
BG = 512
NG = 6
BV = 2048
NV = 8
f32 = jnp.float32


def _ln(x, w, b):
    m = jnp.mean(x, axis=-1, keepdims=True)
    v = jnp.mean((x - m) ** 2, axis=-1, keepdims=True)
    return (x - m) / jnp.sqrt(v + EPS) * w + b


def _mmT(a, w):
    return jax.lax.dot_general(
        a, w, (((1,), (1,)), ((), ())),
        preferred_element_type=f32)


def _embed_body(tok_ref, ti_ref, emb_ref, pos_ref, w_ref, b_ref, o_ref):
    x = emb_ref[...] + pos_ref[...]
    o_ref[...] = _ln(x, w_ref[...], b_ref[...])


def _embed(tok, ti, emb, pos, w, b):
    B = tok.shape[0]
    emb3 = emb.reshape(emb.shape[0], 1, D)
    pos3 = pos.reshape(pos.shape[0], 1, D)
    out = pl.pallas_call(
        _embed_body,
        grid_spec=pltpu.PrefetchScalarGridSpec(
            num_scalar_prefetch=2,
            grid=(B,),
            in_specs=[
                pl.BlockSpec((1, 1, D), lambda i, tok, ti: (tok[i], 0, 0)),
                pl.BlockSpec((1, 1, D), lambda i, tok, ti: (ti[0], 0, 0)),
                pl.BlockSpec((1, D), lambda i, tok, ti: (0, 0)),
                pl.BlockSpec((1, D), lambda i, tok, ti: (0, 0)),
            ],
            out_specs=pl.BlockSpec((1, 1, D), lambda i, tok, ti: (i, 0, 0)),
        ),
        out_shape=jax.ShapeDtypeStruct((B, 1, D), f32),
    )(tok, ti, emb3, pos3, w, b)
    return out.reshape(B, D)


def _sa_body(ti_ref, x_ref, st_ref, wq_ref, wk_ref, wv_ref, wo_ref,
             pw_ref, pb_ref, ow_ref, ob_ref, xo_ref, kv_ref):
    ti = ti_ref[0]
    x = x_ref[...]
    B = x.shape[0]
    h = _ln(x, pw_ref[...], pb_ref[...])
    q = _mmT(h, wq_ref[...])
    k = _mmT(h, wk_ref[...])
    v = _mmT(h, wv_ref[...])
    kv_ref[0:B, :] = k
    kv_ref[B:2 * B, :] = v
    qs = q * (1.0 / (HD ** 0.5))
    K = st_ref[0, 0:B]
    V = st_ref[0, B:2 * B]
    sc_old = (K * qs[:, None, :]).reshape(B, T, H, HD).sum(axis=-1)
    s_new = (k * qs).reshape(B, H, HD).sum(axis=-1)
    tpos = jax.lax.broadcasted_iota(jnp.int32, (B, T, H), 1)
    sc_old = jnp.where(tpos < ti, sc_old, jnp.float32(-1e30))
    m = jnp.maximum(sc_old.max(axis=1), s_new)
    p_old = jnp.exp(sc_old - m[:, None, :])
    p_new = jnp.exp(s_new - m)
    denom = p_old.sum(axis=1) + p_new
    pfull = jnp.broadcast_to(p_old[:, :, :, None], (B, T, H, HD)).reshape(B, T, D)
    ctx = (pfull * V).sum(axis=1)
    ctx = ctx + jnp.broadcast_to(p_new[:, :, None], (B, H, HD)).reshape(B, D) * v
    ctx = ctx / jnp.broadcast_to(denom[:, :, None], (B, H, HD)).reshape(B, D)
    o = _mmT(ctx, wo_ref[...])
    xo_ref[...] = x + _ln(o, ow_ref[...], ob_ref[...])


def _sa(lidx, ti, x, state, wq, wk, wv, wo, pw, pb, ow, ob):
    B = x.shape[0]
    full = lambda _: (0, 0)
    return pl.pallas_call(
        _sa_body,
        grid=(1,),
        in_specs=[
            pl.BlockSpec(memory_space=pltpu.SMEM),
            pl.BlockSpec((B, D), full),
            pl.BlockSpec((1, 2 * B, T, D), lambda _: (lidx, 0, 0, 0)),
            pl.BlockSpec((D, D), full),
            pl.BlockSpec((D, D), full),
            pl.BlockSpec((D, D), full),
            pl.BlockSpec((D, D), full),
            pl.BlockSpec((1, D), full),
            pl.BlockSpec((1, D), full),
            pl.BlockSpec((1, D), full),
            pl.BlockSpec((1, D), full),
        ],
        out_specs=[
            pl.BlockSpec((B, D), full),
            pl.BlockSpec((2 * B, D), full),
        ],
        out_shape=[
            jax.ShapeDtypeStruct((B, D), f32),
            jax.ShapeDtypeStruct((2 * B, D), f32),
        ],
    )(ti, x, state, wq, wk, wv, wo, pw, pb, ow, ob)


def _ca_body(x_ref, enc_ref, mk_ref, wq_ref, wk_ref, wv_ref, wo_ref,
             pw_ref, pb_ref, ow_ref, ob_ref, xo_ref):
    x = x_ref[...]
    B = x.shape[0]
    h = _ln(x, pw_ref[...], pb_ref[...])
    q = _mmT(h, wq_ref[...]) * (1.0 / (HD ** 0.5))
    enc2 = enc_ref[...].reshape(B * S, D)
    K = _mmT(enc2, wk_ref[...]).reshape(B, S, D)
    V = _mmT(enc2, wv_ref[...]).reshape(B, S, D)
    sc = (K * q[:, None, :]).reshape(B, S, H, HD).sum(axis=-1)
    bias = (1.0 - mk_ref[...]) * -1e12
    sc = sc + bias[:, :, None]
    m = sc.max(axis=1)
    p = jnp.exp(sc - m[:, None, :])
    denom = p.sum(axis=1)
    pfull = jnp.broadcast_to(p[:, :, :, None], (B, S, H, HD)).reshape(B, S, D)
    ctx = (pfull * V).sum(axis=1)
    ctx = ctx / jnp.broadcast_to(denom[:, :, None], (B, H, HD)).reshape(B, D)
    o = _mmT(ctx, wo_ref[...])
    xo_ref[...] = x + _ln(o, ow_ref[...], ob_ref[...])


def _ca(x, enc, maskf, wq, wk, wv, wo, pw, pb, ow, ob):
    B = x.shape[0]
    full = lambda _: (0, 0)
    return pl.pallas_call(
        _ca_body,
        grid=(1,),
        in_specs=[
            pl.BlockSpec((B, D), full),
            pl.BlockSpec((B, S, D), lambda _: (0, 0, 0)),
            pl.BlockSpec((B, S), full),
            pl.BlockSpec((D, D), full),
            pl.BlockSpec((D, D), full),
            pl.BlockSpec((D, D), full),
            pl.BlockSpec((D, D), full),
            pl.BlockSpec((1, D), full),
            pl.BlockSpec((1, D), full),
            pl.BlockSpec((1, D), full),
            pl.BlockSpec((1, D), full),
        ],
        out_specs=pl.BlockSpec((B, D), full),
        out_shape=jax.ShapeDtypeStruct((B, D), f32),
    )(x, enc, maskf, wq, wk, wv, wo, pw, pb, ow, ob)


def _glu_a_body(x_ref, w0_ref, b0_ref, fc0_ref, fc1_ref, u_ref):
    z = _ln(x_ref[...], w0_ref[...], b0_ref[...])
    w = _mmT(z, fc0_ref[...])
    wv = _mmT(z, fc1_ref[...])
    gelu = w * 0.5 * (1.0 + jax.lax.erf(w * (2.0 ** -0.5)))
    u_ref[...] = gelu * wv


def _glu_a(x, w0, b0, fc0, fc1):
    B = x.shape[0]
    return pl.pallas_call(
        _glu_a_body,
        grid=(NG,),
        in_specs=[
            pl.BlockSpec((B, D), lambda g: (0, 0)),
            pl.BlockSpec((1, D), lambda g: (0, 0)),
            pl.BlockSpec((1, D), lambda g: (0, 0)),
            pl.BlockSpec((BG, D), lambda g: (g, 0)),
            pl.BlockSpec((BG, D), lambda g: (g, 0)),
        ],
        out_specs=pl.BlockSpec((B, BG), lambda g: (0, g)),
        out_shape=jax.ShapeDtypeStruct((B, G), f32),
    )(x, w0, b0, fc0, fc1)


def _glu_b_body(x_ref, u_ref, w1_ref, b1_ref, fc2_ref, xo_ref):
    z = _ln(u_ref[...], w1_ref[...], b1_ref[...])
    xo_ref[...] = x_ref[...] + _mmT(z, fc2_ref[...])


def _glu_b(x, u, w1, b1, fc2):
    B = x.shape[0]
    full = lambda _: (0, 0)
    return pl.pallas_call(
        _glu_b_body,
        grid=(1,),
        in_specs=[
            pl.BlockSpec((B, D), full),
            pl.BlockSpec((B, G), full),
            pl.BlockSpec((1, G), full),
            pl.BlockSpec((1, G), full),
            pl.BlockSpec((D, G), full),
        ],
        out_specs=pl.BlockSpec((B, D), full),
        out_shape=jax.ShapeDtypeStruct((B, D), f32),
    )(x, u, w1, b1, fc2)


def _lm_body(x_ref, set_ref, fw_ref, fb_ref, w_ref, g_ref, o_ref, acc):
    i = pl.program_id(0)
    zf = _ln(x_ref[...], fw_ref[...], fb_ref[...])
    lb = _mmT(zf, w_ref[...])
    scond = set_ref[2]
    IC = lb.shape[0] // 2
    mixed = lb[0:IC] * (1.0 - scond) + lb[IC:2 * IC] * scond
    acc[:, pl.ds(i * BV, BV)] = mixed

    @pl.when(i == NV - 1)
    def _():
        L = acc[...]
        temp = set_ref[0]
        k = jnp.clip(set_ref[1].astype(jnp.int32), 1, VS)
        top1 = L.max(axis=1, keepdims=True)
        lo0 = L.min(axis=1, keepdims=True)
        hi0 = top1 + jnp.abs(top1) + 1.0

        def body(_, lohi):
            lo, hi = lohi
            mid = lo + (hi - lo) * 0.5
            cnt = (L >= mid).astype(jnp.int32).sum(axis=1, keepdims=True)
            ge = cnt >= k
            return (jnp.where(ge, mid, lo), jnp.where(ge, hi, mid))

        kth, _hi = jax.lax.fori_loop(0, 64, body, (lo0, hi0))
        shifted = (L - top1) / temp
        kept = L >= kth
        probs = jnp.exp(shifted) * kept.astype(f32)
        logp = jnp.where(probs > 0, jnp.log(jnp.maximum(probs, 1e-30)), -1e12)
        score = logp + g_ref[...]
        smax = score.max(axis=1, keepdims=True)
        idx = jax.lax.broadcasted_iota(jnp.int32, score.shape, 1)
        cand = jnp.where(score == smax, idx, VS)
        o_ref[...] = cand.min(axis=1, keepdims=True).reshape(1, -1)


def _lm(x, settings, fw, fb, lm_head, gum):
    B = x.shape[0]
    IC = B // 2
    return pl.pallas_call(
        _lm_body,
        grid=(NV,),
        in_specs=[
            pl.BlockSpec((B, D), lambda i: (0, 0)),
            pl.BlockSpec(memory_space=pltpu.SMEM),
            pl.BlockSpec((1, D), lambda i: (0, 0)),
            pl.BlockSpec((1, D), lambda i: (0, 0)),
            pl.BlockSpec((BV, D), lambda i: (i, 0)),
            pl.BlockSpec((IC, VS), lambda i: (0, 0)),
        ],
        out_specs=pl.BlockSpec((1, IC), lambda i: (0, 0)),
        out_shape=jax.ShapeDtypeStruct((1, IC), jnp.int32),
        scratch_shapes=[pltpu.VMEM((IC, VS), f32)],
    )(x, settings, fw, fb, lm_head, gum)


def _stupd_body(ti_ref, st_ref, kv_ref, o_ref):
    r = ti_ref[0] % 8
    o_ref[...] = st_ref[...]
    o_ref[0, :, pl.ds(r, 1), :] = kv_ref[...][0, :, None, :]


def _state_update(ti, state, kv):
    B2 = kv.shape[1]
    return pl.pallas_call(
        _stupd_body,
        grid_spec=pltpu.PrefetchScalarGridSpec(
            num_scalar_prefetch=1,
            grid=(DEPTH,),
            in_specs=[
                pl.BlockSpec((1, B2, 8, D), lambda l, ti: (l, 0, ti[0] // 8, 0)),
                pl.BlockSpec((1, B2, D), lambda l, ti: (l, 0, 0)),
            ],
            out_specs=pl.BlockSpec((1, B2, 8, D), lambda l, ti: (l, 0, ti[0] // 8, 0)),
        ),
        out_shape=jax.ShapeDtypeStruct(state.shape, state.dtype),
        input_output_aliases={1: 0},
    )(ti, state, kv)


def kernel(settings, attention_mask, encoder_state, attention_state,
           prev_tokens, token_index, params):
    IC = encoder_state.shape[0] // 2
    ti = token_index.astype(jnp.int32)
    tok = jnp.clip(jnp.concatenate([prev_tokens, prev_tokens]), 0,
                   params['embed_tokens'].shape[0] - 1).astype(jnp.int32)
    r2 = lambda a: a.reshape(1, -1)
    x = _embed(tok, ti, params['embed_tokens'], params['embed_positions'],
               r2(params['ln_emb_w']), r2(params['ln_emb_b']))
    maskf = attention_mask.astype(f32)
    kvs = []
    for l in range(DEPTH):
        lp = params['layers'][l]
        x, kv = _sa(l, ti, x, attention_state,
                    lp['sa_q'], lp['sa_k'], lp['sa_v'], lp['sa_o'],
                    r2(lp['pre_sa_ln_w']), r2(lp['pre_sa_ln_b']),
                    r2(lp['sa_ln_w']), r2(lp['sa_ln_b']))
        kvs.append(kv)
        x = _ca(x, encoder_state, maskf,
                lp['ca_q'], lp['ca_k'], lp['ca_v'], lp['ca_o'],
                r2(lp['pre_ca_ln_w']), r2(lp['pre_ca_ln_b']),
                r2(lp['ca_ln_w']), r2(lp['ca_ln_b']))
        u = _glu_a(x, r2(lp['glu_ln0_w']), r2(lp['glu_ln0_b']),
                   lp['glu_fc0'], lp['glu_fc1'])
        x = _glu_b(x, u, r2(lp['glu_ln1_w']), r2(lp['glu_ln1_b']),
                   lp['glu_fc2'])
    kvst = jnp.stack(kvs, axis=0)
    new_state = _state_update(ti, attention_state, kvst)
    gum = jax.random.gumbel(jax.random.key(42), (IC, VS), f32)
    tokens = _lm(x, settings, r2(params['final_ln_w']),
                 r2(params['final_ln_b']), params['lm_head'], gum)
    return tokens.reshape(IC), new_state
